# baseline (speedup 1.0000x reference)
import functools

import jax
import jax.numpy as jnp
from jax import lax
from jax.experimental import pallas as pl
from jax.experimental.pallas import tpu as pltpu
from jax.experimental.pallas import tpu_sc as plsc

N = 10000
D = 128
NC = 2
NS = 16
NW = NC * NS
CHUNK = 128
NPAD = 10240
ROWS_PER_TILE = NPAD // NS
TRASH = N


def _deg_body(cpw0, cpw1, col_hbm, degp_hbm, col_v, ones_v, zb_v, deg_s):
    c = lax.axis_index("c")
    s = lax.axis_index("s")
    nch = jnp.where(c == 0, cpw0, cpw1)

    def fill(i, _):
        zb_v[pl.ds(i * 16, 16)] = jnp.zeros((16,), jnp.float32)
        return 0
    lax.fori_loop(0, ROWS_PER_TILE // 16, fill, 0)

    def fill1(i, _):
        ones_v[pl.ds(i * 16, 16)] = jnp.ones((16,), jnp.float32)
        return 0
    lax.fori_loop(0, CHUNK // 16, fill1, 0)

    pltpu.sync_copy(zb_v, deg_s.at[pl.ds(s * ROWS_PER_TILE, ROWS_PER_TILE)])
    plsc.subcore_barrier()

    pltpu.sync_copy(col_hbm.at[c, s], col_v)

    def step(j, _):
        pltpu.sync_copy(ones_v, deg_s.at[col_v.at[j]], add=True)
        return 0
    lax.fori_loop(0, nch, step, 0)

    plsc.subcore_barrier()
    pltpu.sync_copy(deg_s.at[pl.ds(s * ROWS_PER_TILE, ROWS_PER_TILE)],
                    degp_hbm.at[c, pl.ds(s * ROWS_PER_TILE, ROWS_PER_TILE)])


def _agg_body(cpw0, cpw1, h_hbm, row_hbm, col_hbm, aggp_hbm,
              rowi_v, coli_v, rows_v, agg_s):
    c = lax.axis_index("c")
    s = lax.axis_index("s")
    nch = jnp.where(c == 0, cpw0, cpw1)

    def fill(i, _):
        for k in range(8):
            rows_v[i, pl.ds(k * 16, 16)] = jnp.zeros((16,), jnp.float32)
        return 0
    lax.fori_loop(0, 128, fill, 0)
    for k in range(ROWS_PER_TILE // 128):
        pltpu.sync_copy(rows_v, agg_s.at[pl.ds(s * ROWS_PER_TILE + k * 128, 128)])
    plsc.subcore_barrier()

    pltpu.sync_copy(row_hbm.at[c, s], rowi_v)
    pltpu.sync_copy(col_hbm.at[c, s], coli_v)

    def step(j, _):
        pltpu.sync_copy(h_hbm.at[rowi_v.at[j]], rows_v)
        pltpu.sync_copy(rows_v, agg_s.at[coli_v.at[j]], add=True)
        return 0
    lax.fori_loop(0, nch, step, 0)

    plsc.subcore_barrier()
    pltpu.sync_copy(agg_s.at[pl.ds(s * ROWS_PER_TILE, ROWS_PER_TILE)],
                    aggp_hbm.at[c, pl.ds(s * ROWS_PER_TILE, ROWS_PER_TILE)])


def _sc_degree(col4d, cpw0, cpw1):
    body = functools.partial(_deg_body, cpw0, cpw1)
    return pl.kernel(
        body,
        out_type=jax.ShapeDtypeStruct((NC, NPAD), jnp.float32),
        mesh=plsc.VectorSubcoreMesh(core_axis_name="c", subcore_axis_name="s"),
        scratch_types=[
            pltpu.VMEM((cpw0, CHUNK), jnp.int32),
            pltpu.VMEM((CHUNK,), jnp.float32),
            pltpu.VMEM((ROWS_PER_TILE,), jnp.float32),
            pltpu.VMEM_SHARED((NPAD,), jnp.float32),
        ],
    )(col4d)


def _sc_aggregate(h, row4d, col4d, cpw0, cpw1):
    body = functools.partial(_agg_body, cpw0, cpw1)
    return pl.kernel(
        body,
        out_type=jax.ShapeDtypeStruct((NC, NPAD, D), jnp.float32),
        mesh=plsc.VectorSubcoreMesh(core_axis_name="c", subcore_axis_name="s"),
        scratch_types=[
            pltpu.VMEM((cpw0, CHUNK), jnp.int32),
            pltpu.VMEM((cpw0, CHUNK), jnp.int32),
            pltpu.VMEM((CHUNK, D), jnp.float32),
            pltpu.VMEM_SHARED((NPAD, D), jnp.float32),
        ],
    )(h, row4d, col4d)


_BM = 400


def _tc1_body(x_ref, w_ref, degp_ref, o_ref):
    dis = lax.rsqrt(degp_ref[0] + degp_ref[1] + 1.0)
    h = lax.dot_general(x_ref[...], w_ref[...], (((1,), (1,)), ((), ())),
                        preferred_element_type=jnp.float32)
    o_ref[...] = h * dis


def _tc_scale_matmul(x, w, degp):
    return pl.pallas_call(
        _tc1_body,
        grid=(N // _BM,),
        in_specs=[
            pl.BlockSpec((_BM, D), lambda i: (i, 0)),
            pl.BlockSpec((D, D), lambda i: (0, 0)),
            pl.BlockSpec((NC, _BM, 1), lambda i: (0, i, 0)),
        ],
        out_specs=pl.BlockSpec((_BM, D), lambda i: (i, 0)),
        out_shape=jax.ShapeDtypeStruct((N, D), jnp.float32),
    )(x, w, degp)


def _tc2_body(aggp_ref, hp_ref, degp_ref, b_ref, w_ref, o_ref):
    dis = lax.rsqrt(degp_ref[0] + degp_ref[1] + 1.0)
    agg = aggp_ref[0] + aggp_ref[1] + hp_ref[...]
    h = dis * agg + b_ref[...]
    h = jnp.where(h >= 0, h, 0.01 * h)
    h2 = lax.dot_general(h, w_ref[...], (((1,), (1,)), ((), ())),
                         preferred_element_type=jnp.float32)
    o_ref[...] = h2 * dis


def _tc_combine_matmul(aggp, hp, degp, b, w):
    return pl.pallas_call(
        _tc2_body,
        grid=(N // _BM,),
        in_specs=[
            pl.BlockSpec((NC, _BM, D), lambda i: (0, i, 0)),
            pl.BlockSpec((_BM, D), lambda i: (i, 0)),
            pl.BlockSpec((NC, _BM, 1), lambda i: (0, i, 0)),
            pl.BlockSpec((1, D), lambda i: (0, 0)),
            pl.BlockSpec((D, D), lambda i: (0, 0)),
        ],
        out_specs=pl.BlockSpec((_BM, D), lambda i: (i, 0)),
        out_shape=jax.ShapeDtypeStruct((N, D), jnp.float32),
    )(aggp, hp, degp, b, w)


def _tc3_body(aggp_ref, hp_ref, degp_ref, b_ref, o_ref):
    dis = lax.rsqrt(degp_ref[0] + degp_ref[1] + 1.0)
    agg = aggp_ref[0] + aggp_ref[1] + hp_ref[...]
    h = dis * agg + b_ref[...]
    o_ref[...] = jnp.where(h >= 0, h, 0.01 * h)


def _tc_combine_final(aggp, hp, degp, b):
    return pl.pallas_call(
        _tc3_body,
        grid=(N // _BM,),
        in_specs=[
            pl.BlockSpec((NC, _BM, D), lambda i: (0, i, 0)),
            pl.BlockSpec((_BM, D), lambda i: (i, 0)),
            pl.BlockSpec((NC, _BM, 1), lambda i: (0, i, 0)),
            pl.BlockSpec((1, D), lambda i: (0, 0)),
        ],
        out_specs=pl.BlockSpec((_BM, D), lambda i: (i, 0)),
        out_shape=jax.ShapeDtypeStruct((N, D), jnp.float32),
    )(aggp, hp, degp, b)


def kernel(x, a, W0, b0, W1, b1):
    E = a.shape[1]
    cpt = -(-E // (CHUNK * NS))
    cpw0 = (cpt * 54 + 99) // 100
    cpw1 = cpt - cpw0
    e0 = NS * cpw0 * CHUNK
    e1 = NS * cpw1 * CHUNK
    pad = e0 + e1 - E

    row = jnp.concatenate([a[0], jnp.zeros((pad,), jnp.int32)])
    col = jnp.concatenate([a[1], jnp.full((pad,), TRASH, jnp.int32)])
    padc = ((0, 0), (0, cpw0 - cpw1), (0, 0))
    row4d = jnp.stack([
        row[:e0].reshape(NS, cpw0, CHUNK),
        jnp.pad(row[e0:].reshape(NS, cpw1, CHUNK), padc),
    ])
    col4d = jnp.stack([
        col[:e0].reshape(NS, cpw0, CHUNK),
        jnp.pad(col[e0:].reshape(NS, cpw1, CHUNK), padc, constant_values=TRASH),
    ])

    degp = _sc_degree(col4d, cpw0, cpw1)
    degp = degp.reshape(NC, NPAD, 1)

    h0 = _tc_scale_matmul(x, W0, degp)
    agg0 = _sc_aggregate(h0, row4d, col4d, cpw0, cpw1)
    h1 = _tc_combine_matmul(agg0, h0, degp,
                            b0.reshape(1, D), W1)
    agg1 = _sc_aggregate(h1, row4d, col4d, cpw0, cpw1)
    out = _tc_combine_final(agg1, h1, degp, b1.reshape(1, D))
    return out


# speedup vs baseline: 1.0503x; 1.0503x over previous
import functools

import jax
import jax.numpy as jnp
from jax import lax
from jax.experimental import pallas as pl
from jax.experimental.pallas import tpu as pltpu
from jax.experimental.pallas import tpu_sc as plsc

N = 10000
D = 128
NC = 2
NS = 16
NW = NC * NS
CHUNK = 128
NPAD = 10240
ROWS_PER_TILE = NPAD // NS
TRASH = N


def _deg_body(cpw0, cpw1, col_hbm, degp_hbm, col_v, ones_v, zb_v, deg_s):
    c = lax.axis_index("c")
    s = lax.axis_index("s")
    nch = jnp.where(c == 0, cpw0, cpw1)

    def fill(i, _):
        zb_v[pl.ds(i * 16, 16)] = jnp.zeros((16,), jnp.float32)
        return 0
    lax.fori_loop(0, ROWS_PER_TILE // 16, fill, 0)

    def fill1(i, _):
        ones_v[pl.ds(i * 16, 16)] = jnp.ones((16,), jnp.float32)
        return 0
    lax.fori_loop(0, CHUNK // 16, fill1, 0)

    pltpu.sync_copy(zb_v, deg_s.at[pl.ds(s * ROWS_PER_TILE, ROWS_PER_TILE)])
    plsc.subcore_barrier()

    pltpu.sync_copy(col_hbm.at[c, s], col_v)

    def step(j, _):
        pltpu.sync_copy(ones_v, deg_s.at[col_v.at[j]], add=True)
        return 0
    lax.fori_loop(0, nch, step, 0)

    plsc.subcore_barrier()
    pltpu.sync_copy(deg_s.at[pl.ds(s * ROWS_PER_TILE, ROWS_PER_TILE)],
                    degp_hbm.at[c, pl.ds(s * ROWS_PER_TILE, ROWS_PER_TILE)])


def _agg_body(cpw0, cpw1, h_hbm, row_hbm, col_hbm, aggp_hbm,
              rowi_v, coli_v, rows_v, agg_s):
    c = lax.axis_index("c")
    s = lax.axis_index("s")
    nch = jnp.where(c == 0, cpw0, cpw1)

    def fill(i, _):
        for k in range(8):
            rows_v[i, pl.ds(k * 16, 16)] = jnp.zeros((16,), jnp.float32)
        return 0
    lax.fori_loop(0, 128, fill, 0)
    for k in range(ROWS_PER_TILE // 128):
        pltpu.sync_copy(rows_v, agg_s.at[pl.ds(s * ROWS_PER_TILE + k * 128, 128)])
    plsc.subcore_barrier()

    pltpu.sync_copy(row_hbm.at[c, s], rowi_v)
    pltpu.sync_copy(col_hbm.at[c, s], coli_v)

    def step(j, _):
        pltpu.sync_copy(h_hbm.at[rowi_v.at[j]], rows_v)
        pltpu.sync_copy(rows_v, agg_s.at[coli_v.at[j]], add=True)
        return 0
    lax.fori_loop(0, nch, step, 0)

    plsc.subcore_barrier()
    pltpu.sync_copy(agg_s.at[pl.ds(s * ROWS_PER_TILE, ROWS_PER_TILE)],
                    aggp_hbm.at[c, pl.ds(s * ROWS_PER_TILE, ROWS_PER_TILE)])


def _sc_degree(col4d, cpw0, cpw1):
    body = functools.partial(_deg_body, cpw0, cpw1)
    return pl.kernel(
        body,
        out_type=jax.ShapeDtypeStruct((NC, NPAD), jnp.float32),
        mesh=plsc.VectorSubcoreMesh(core_axis_name="c", subcore_axis_name="s"),
        scratch_types=[
            pltpu.VMEM((cpw0, CHUNK), jnp.int32),
            pltpu.VMEM((CHUNK,), jnp.float32),
            pltpu.VMEM((ROWS_PER_TILE,), jnp.float32),
            pltpu.VMEM_SHARED((NPAD,), jnp.float32),
        ],
    )(col4d)


def _sc_aggregate(h, row4d, col4d, cpw0, cpw1):
    body = functools.partial(_agg_body, cpw0, cpw1)
    return pl.kernel(
        body,
        out_type=jax.ShapeDtypeStruct((NC, NPAD, D), jnp.float32),
        mesh=plsc.VectorSubcoreMesh(core_axis_name="c", subcore_axis_name="s"),
        scratch_types=[
            pltpu.VMEM((cpw0, CHUNK), jnp.int32),
            pltpu.VMEM((cpw0, CHUNK), jnp.int32),
            pltpu.VMEM((CHUNK, D), jnp.float32),
            pltpu.VMEM_SHARED((NPAD, D), jnp.float32),
        ],
    )(h, row4d, col4d)


_BM = 400


def _tc1_body(x_ref, w_ref, degp_ref, o_ref):
    dis = lax.rsqrt(degp_ref[0] + degp_ref[1] + 1.0)
    h = lax.dot_general(x_ref[...], w_ref[...], (((1,), (1,)), ((), ())),
                        preferred_element_type=jnp.float32)
    o_ref[...] = h * dis


def _tc_scale_matmul(x, w, degp):
    return pl.pallas_call(
        _tc1_body,
        grid=(N // _BM,),
        in_specs=[
            pl.BlockSpec((_BM, D), lambda i: (i, 0)),
            pl.BlockSpec((D, D), lambda i: (0, 0)),
            pl.BlockSpec((NC, _BM, 1), lambda i: (0, i, 0)),
        ],
        out_specs=pl.BlockSpec((_BM, D), lambda i: (i, 0)),
        out_shape=jax.ShapeDtypeStruct((N, D), jnp.float32),
    )(x, w, degp)


def _tc2_body(aggp_ref, hp_ref, degp_ref, b_ref, w_ref, o_ref):
    dis = lax.rsqrt(degp_ref[0] + degp_ref[1] + 1.0)
    agg = aggp_ref[0] + aggp_ref[1] + hp_ref[...]
    h = dis * agg + b_ref[...]
    h = jnp.where(h >= 0, h, 0.01 * h)
    h2 = lax.dot_general(h, w_ref[...], (((1,), (1,)), ((), ())),
                         preferred_element_type=jnp.float32)
    o_ref[...] = h2 * dis


def _tc_combine_matmul(aggp, hp, degp, b, w):
    return pl.pallas_call(
        _tc2_body,
        grid=(N // _BM,),
        in_specs=[
            pl.BlockSpec((NC, _BM, D), lambda i: (0, i, 0)),
            pl.BlockSpec((_BM, D), lambda i: (i, 0)),
            pl.BlockSpec((NC, _BM, 1), lambda i: (0, i, 0)),
            pl.BlockSpec((1, D), lambda i: (0, 0)),
            pl.BlockSpec((D, D), lambda i: (0, 0)),
        ],
        out_specs=pl.BlockSpec((_BM, D), lambda i: (i, 0)),
        out_shape=jax.ShapeDtypeStruct((N, D), jnp.float32),
    )(aggp, hp, degp, b, w)


def _tc3_body(aggp_ref, hp_ref, degp_ref, b_ref, o_ref):
    dis = lax.rsqrt(degp_ref[0] + degp_ref[1] + 1.0)
    agg = aggp_ref[0] + aggp_ref[1] + hp_ref[...]
    h = dis * agg + b_ref[...]
    o_ref[...] = jnp.where(h >= 0, h, 0.01 * h)


def _tc_combine_final(aggp, hp, degp, b):
    return pl.pallas_call(
        _tc3_body,
        grid=(N // _BM,),
        in_specs=[
            pl.BlockSpec((NC, _BM, D), lambda i: (0, i, 0)),
            pl.BlockSpec((_BM, D), lambda i: (i, 0)),
            pl.BlockSpec((NC, _BM, 1), lambda i: (0, i, 0)),
            pl.BlockSpec((1, D), lambda i: (0, 0)),
        ],
        out_specs=pl.BlockSpec((_BM, D), lambda i: (i, 0)),
        out_shape=jax.ShapeDtypeStruct((N, D), jnp.float32),
    )(aggp, hp, degp, b)


def kernel(x, a, W0, b0, W1, b1):
    E = a.shape[1]
    cpt = -(-E // (CHUNK * NS))
    cpw0 = (cpt * 57 + 99) // 100
    cpw1 = cpt - cpw0
    e0 = NS * cpw0 * CHUNK
    e1 = NS * cpw1 * CHUNK
    pad = e0 + e1 - E

    row = jnp.concatenate([a[0], jnp.zeros((pad,), jnp.int32)])
    col = jnp.concatenate([a[1], jnp.full((pad,), TRASH, jnp.int32)])
    padc = ((0, 0), (0, cpw0 - cpw1), (0, 0))
    row4d = jnp.stack([
        row[:e0].reshape(NS, cpw0, CHUNK),
        jnp.pad(row[e0:].reshape(NS, cpw1, CHUNK), padc),
    ])
    col4d = jnp.stack([
        col[:e0].reshape(NS, cpw0, CHUNK),
        jnp.pad(col[e0:].reshape(NS, cpw1, CHUNK), padc, constant_values=TRASH),
    ])

    degp = _sc_degree(col4d, cpw0, cpw1)
    degp = degp.reshape(NC, NPAD, 1)

    h0 = _tc_scale_matmul(x, W0, degp)
    agg0 = _sc_aggregate(h0, row4d, col4d, cpw0, cpw1)
    h1 = _tc_combine_matmul(agg0, h0, degp,
                            b0.reshape(1, D), W1)
    agg1 = _sc_aggregate(h1, row4d, col4d, cpw0, cpw1)
    out = _tc_combine_final(agg1, h1, degp, b1.reshape(1, D))
    return out


# speedup vs baseline: 1.0641x; 1.0132x over previous
import functools

import jax
import jax.numpy as jnp
from jax import lax
from jax.experimental import pallas as pl
from jax.experimental.pallas import tpu as pltpu
from jax.experimental.pallas import tpu_sc as plsc

N = 10000
D = 128
NC = 2
NS = 16
NW = NC * NS
CHUNK = 128
NPAD = 10240
ROWS_PER_TILE = NPAD // NS
TRASH = N


def _deg_body(cpw0, cpw1, col0_hbm, col1_hbm, degp_hbm, col_v, ones_v, zb_v,
              deg_s):
    c = lax.axis_index("c")
    s = lax.axis_index("s")
    nch = jnp.where(c == 0, cpw0, cpw1)

    def fill(i, _):
        zb_v[pl.ds(i * 16, 16)] = jnp.zeros((16,), jnp.float32)
        return 0
    lax.fori_loop(0, ROWS_PER_TILE // 16, fill, 0)

    def fill1(i, _):
        ones_v[pl.ds(i * 16, 16)] = jnp.ones((16,), jnp.float32)
        return 0
    lax.fori_loop(0, CHUNK // 16, fill1, 0)

    pltpu.sync_copy(zb_v, deg_s.at[pl.ds(s * ROWS_PER_TILE, ROWS_PER_TILE)])
    plsc.subcore_barrier()

    @pl.when(c == 0)
    def _():
        pltpu.sync_copy(col0_hbm.at[s], col_v)

    @pl.when(c == 1)
    def _():
        pltpu.sync_copy(col1_hbm.at[s], col_v.at[pl.ds(0, cpw1)])

    def step(j, _):
        pltpu.sync_copy(ones_v, deg_s.at[col_v.at[j]], add=True)
        return 0
    lax.fori_loop(0, nch, step, 0)

    plsc.subcore_barrier()
    pltpu.sync_copy(deg_s.at[pl.ds(s * ROWS_PER_TILE, ROWS_PER_TILE)],
                    degp_hbm.at[c, pl.ds(s * ROWS_PER_TILE, ROWS_PER_TILE)])


def _agg_body(cpw0, cpw1, h_hbm, row0_hbm, row1_hbm, col0_hbm, col1_hbm,
              aggp_hbm, rowi_v, coli_v, rows_v, agg_s):
    c = lax.axis_index("c")
    s = lax.axis_index("s")
    nch = jnp.where(c == 0, cpw0, cpw1)

    def fill(i, _):
        for k in range(8):
            rows_v[i, pl.ds(k * 16, 16)] = jnp.zeros((16,), jnp.float32)
        return 0
    lax.fori_loop(0, 128, fill, 0)
    for k in range(ROWS_PER_TILE // 128):
        pltpu.sync_copy(rows_v, agg_s.at[pl.ds(s * ROWS_PER_TILE + k * 128, 128)])
    plsc.subcore_barrier()

    @pl.when(c == 0)
    def _():
        pltpu.sync_copy(row0_hbm.at[s], rowi_v)
        pltpu.sync_copy(col0_hbm.at[s], coli_v)

    @pl.when(c == 1)
    def _():
        pltpu.sync_copy(row1_hbm.at[s], rowi_v.at[pl.ds(0, cpw1)])
        pltpu.sync_copy(col1_hbm.at[s], coli_v.at[pl.ds(0, cpw1)])

    def step(j, _):
        pltpu.sync_copy(h_hbm.at[rowi_v.at[j]], rows_v)
        pltpu.sync_copy(rows_v, agg_s.at[coli_v.at[j]], add=True)
        return 0
    lax.fori_loop(0, nch, step, 0)

    plsc.subcore_barrier()
    pltpu.sync_copy(agg_s.at[pl.ds(s * ROWS_PER_TILE, ROWS_PER_TILE)],
                    aggp_hbm.at[c, pl.ds(s * ROWS_PER_TILE, ROWS_PER_TILE)])


def _sc_degree(col0, col1, cpw0, cpw1):
    body = functools.partial(_deg_body, cpw0, cpw1)
    return pl.kernel(
        body,
        out_type=jax.ShapeDtypeStruct((NC, NPAD), jnp.float32),
        mesh=plsc.VectorSubcoreMesh(core_axis_name="c", subcore_axis_name="s"),
        scratch_types=[
            pltpu.VMEM((cpw0, CHUNK), jnp.int32),
            pltpu.VMEM((CHUNK,), jnp.float32),
            pltpu.VMEM((ROWS_PER_TILE,), jnp.float32),
            pltpu.VMEM_SHARED((NPAD,), jnp.float32),
        ],
    )(col0, col1)


def _sc_aggregate(h, row0, row1, col0, col1, cpw0, cpw1):
    body = functools.partial(_agg_body, cpw0, cpw1)
    return pl.kernel(
        body,
        out_type=jax.ShapeDtypeStruct((NC, NPAD, D), jnp.float32),
        mesh=plsc.VectorSubcoreMesh(core_axis_name="c", subcore_axis_name="s"),
        scratch_types=[
            pltpu.VMEM((cpw0, CHUNK), jnp.int32),
            pltpu.VMEM((cpw0, CHUNK), jnp.int32),
            pltpu.VMEM((CHUNK, D), jnp.float32),
            pltpu.VMEM_SHARED((NPAD, D), jnp.float32),
        ],
    )(h, row0, row1, col0, col1)


_BM = 2000


def _tc1_body(x_ref, w_ref, degp_ref, o_ref):
    dis = lax.rsqrt(degp_ref[0] + degp_ref[1] + 1.0)
    h = lax.dot_general(x_ref[...], w_ref[...], (((1,), (1,)), ((), ())),
                        preferred_element_type=jnp.float32)
    o_ref[...] = h * dis


def _tc_scale_matmul(x, w, degp):
    return pl.pallas_call(
        _tc1_body,
        grid=(N // _BM,),
        in_specs=[
            pl.BlockSpec((_BM, D), lambda i: (i, 0)),
            pl.BlockSpec((D, D), lambda i: (0, 0)),
            pl.BlockSpec((NC, _BM, 1), lambda i: (0, i, 0)),
        ],
        out_specs=pl.BlockSpec((_BM, D), lambda i: (i, 0)),
        out_shape=jax.ShapeDtypeStruct((N, D), jnp.float32),
    )(x, w, degp)


def _tc2_body(aggp_ref, hp_ref, degp_ref, b_ref, w_ref, o_ref):
    dis = lax.rsqrt(degp_ref[0] + degp_ref[1] + 1.0)
    agg = aggp_ref[0] + aggp_ref[1] + hp_ref[...]
    h = dis * agg + b_ref[...]
    h = jnp.where(h >= 0, h, 0.01 * h)
    h2 = lax.dot_general(h, w_ref[...], (((1,), (1,)), ((), ())),
                         preferred_element_type=jnp.float32)
    o_ref[...] = h2 * dis


def _tc_combine_matmul(aggp, hp, degp, b, w):
    return pl.pallas_call(
        _tc2_body,
        grid=(N // _BM,),
        in_specs=[
            pl.BlockSpec((NC, _BM, D), lambda i: (0, i, 0)),
            pl.BlockSpec((_BM, D), lambda i: (i, 0)),
            pl.BlockSpec((NC, _BM, 1), lambda i: (0, i, 0)),
            pl.BlockSpec((1, D), lambda i: (0, 0)),
            pl.BlockSpec((D, D), lambda i: (0, 0)),
        ],
        out_specs=pl.BlockSpec((_BM, D), lambda i: (i, 0)),
        out_shape=jax.ShapeDtypeStruct((N, D), jnp.float32),
    )(aggp, hp, degp, b, w)


def _tc3_body(aggp_ref, hp_ref, degp_ref, b_ref, o_ref):
    dis = lax.rsqrt(degp_ref[0] + degp_ref[1] + 1.0)
    agg = aggp_ref[0] + aggp_ref[1] + hp_ref[...]
    h = dis * agg + b_ref[...]
    o_ref[...] = jnp.where(h >= 0, h, 0.01 * h)


def _tc_combine_final(aggp, hp, degp, b):
    return pl.pallas_call(
        _tc3_body,
        grid=(N // _BM,),
        in_specs=[
            pl.BlockSpec((NC, _BM, D), lambda i: (0, i, 0)),
            pl.BlockSpec((_BM, D), lambda i: (i, 0)),
            pl.BlockSpec((NC, _BM, 1), lambda i: (0, i, 0)),
            pl.BlockSpec((1, D), lambda i: (0, 0)),
        ],
        out_specs=pl.BlockSpec((_BM, D), lambda i: (i, 0)),
        out_shape=jax.ShapeDtypeStruct((N, D), jnp.float32),
    )(aggp, hp, degp, b)


def kernel(x, a, W0, b0, W1, b1):
    E = a.shape[1]
    cpt = -(-E // (CHUNK * NS))
    cpw0 = (cpt * 57 + 99) // 100
    cpw1 = cpt - cpw0
    e0 = NS * cpw0 * CHUNK
    e1 = NS * cpw1 * CHUNK
    pad = e0 + e1 - E

    row = jnp.concatenate([a[0], jnp.zeros((pad,), jnp.int32)])
    col = jnp.concatenate([a[1], jnp.full((pad,), TRASH, jnp.int32)])
    row0 = row[:e0].reshape(NS, cpw0, CHUNK)
    row1 = row[e0:].reshape(NS, cpw1, CHUNK)
    col0 = col[:e0].reshape(NS, cpw0, CHUNK)
    col1 = col[e0:].reshape(NS, cpw1, CHUNK)

    degp = _sc_degree(col0, col1, cpw0, cpw1)
    degp = degp.reshape(NC, NPAD, 1)

    h0 = _tc_scale_matmul(x, W0, degp)
    agg0 = _sc_aggregate(h0, row0, row1, col0, col1, cpw0, cpw1)
    h1 = _tc_combine_matmul(agg0, h0, degp,
                            b0.reshape(1, D), W1)
    agg1 = _sc_aggregate(h1, row0, row1, col0, col1, cpw0, cpw1)
    out = _tc_combine_final(agg1, h1, degp, b1.reshape(1, D))
    return out
